# Initial kernel scaffold; baseline (speedup 1.0000x reference)
import jax
import jax.numpy as jnp
from jax.experimental import pallas as pl


def kernel(x, record_len):
    raise NotImplementedError("write your pallas kernel here")


# speedup vs baseline: 4.5808x; 4.5808x over previous
import functools

import jax
import jax.numpy as jnp
from jax import lax
from jax.experimental import pallas as pl
from jax.experimental.pallas import tpu as pltpu
from jax.experimental.pallas import tpu_sc as plsc

N = 32768
D = 256
B = 16
L = 16
NC = 2
NS = 16
NW = NC * NS
ROWS_PER_W = N // NW
CHUNK = 256
NCHUNK = ROWS_PER_W // CHUNK
NEG = float("-inf")


def _seg_of(cum, g):
    cnt = jnp.sum(jnp.where(cum <= g, 1, 0).astype(jnp.int32))
    return jnp.minimum(cnt, B - 1)


def _lane_pick(vec, s):
    lane = lax.iota(jnp.int32, L)
    return jnp.max(jnp.where(lane == s, vec, -1))


def _sc_body(x_hbm, rl_hbm, out_hbm, rl_v, buf, acc, tmp, res, shared):
    cid = lax.axis_index("c")
    sid = lax.axis_index("s")
    wid = cid * NS + sid
    row0 = wid * ROWS_PER_W

    pltpu.sync_copy(rl_hbm, rl_v)
    rl = rl_v[...]
    cum = jnp.cumsum(rl)
    seg_start = cum - rl
    lane = lax.iota(jnp.int32, L)
    seg_end = jnp.where(lane == B - 1, N, cum)

    neg = jnp.full((L,), NEG, jnp.float32)
    for s in range(B):
        for j in range(D // L):
            acc[s, pl.ds(j * L, L)] = neg

    for c in range(NCHUNK):
        cb = row0 + c * CHUNK
        pltpu.sync_copy(x_hbm.at[pl.ds(cb, CHUNK)], buf)
        s_lo = _seg_of(cum, cb)
        s_hi = _seg_of(cum, cb + CHUNK - 1)

        def seg_body(s, _, cb=cb):
            lo = jnp.maximum(_lane_pick(seg_start, s), cb) - cb
            hi = jnp.minimum(_lane_pick(seg_end, s), cb + CHUNK) - cb

            def row_body(r, accs):
                return tuple(
                    jnp.maximum(accs[j], buf[r, pl.ds(j * L, L)])
                    for j in range(D // L)
                )

            accs = lax.fori_loop(lo, hi, row_body, (neg,) * (D // L))
            rows_idx = jnp.full((L,), s, jnp.int32)
            for j in range(D // L):
                cols = lane + j * L
                old = plsc.load_gather(acc, [rows_idx, cols])
                plsc.store_scatter(
                    acc, [rows_idx, cols], jnp.maximum(old, accs[j])
                )
            return 0

        lax.fori_loop(s_lo, s_hi + 1, seg_body, 0)

    pltpu.sync_copy(acc, shared.at[sid])
    plsc.subcore_barrier()
    for t in range(NS):
        pltpu.sync_copy(shared.at[t, sid], tmp.at[t])
    for j in range(D // L):
        m = tmp[0, pl.ds(j * L, L)]
        for t in range(1, NS):
            m = jnp.maximum(m, tmp[t, pl.ds(j * L, L)])
        res[pl.ds(j * L, L)] = m
    pltpu.sync_copy(res, out_hbm.at[cid, sid])


@jax.jit
def kernel(x, record_len):
    xs = jnp.reshape(x, (N, D))
    mesh = plsc.VectorSubcoreMesh(core_axis_name="c", subcore_axis_name="s")
    f = functools.partial(
        pl.kernel,
        out_type=jax.ShapeDtypeStruct((NC, B, D), jnp.float32),
        mesh=mesh,
        compiler_params=pltpu.CompilerParams(needs_layout_passes=False),
        scratch_types=[
            pltpu.VMEM((B,), jnp.int32),
            pltpu.VMEM((CHUNK, D), jnp.float32),
            pltpu.VMEM((B, D), jnp.float32),
            pltpu.VMEM((NS, D), jnp.float32),
            pltpu.VMEM((D,), jnp.float32),
            pltpu.VMEM_SHARED((NS, B, D), jnp.float32),
        ],
    )(_sc_body)
    part = f(xs, record_len)
    return jnp.maximum(part[0], part[1])


# speedup vs baseline: 4.7844x; 1.0444x over previous
import functools

import jax
import jax.numpy as jnp
from jax import lax
from jax.experimental import pallas as pl
from jax.experimental.pallas import tpu as pltpu
from jax.experimental.pallas import tpu_sc as plsc

N = 32768
D = 256
B = 16
L = 16
NC = 2
NS = 16
NW = NC * NS
ROWS_PER_W = N // NW
CHUNK = 128
NCHUNK = ROWS_PER_W // CHUNK
NEG = float("-inf")


def _seg_of(cum, g):
    cnt = jnp.sum(jnp.where(cum <= g, 1, 0).astype(jnp.int32))
    return jnp.minimum(cnt, B - 1)


def _lane_pick(vec, s):
    lane = lax.iota(jnp.int32, L)
    return jnp.max(jnp.where(lane == s, vec, -1))


def _sc_body(x_hbm, rl_hbm, out_hbm, rl_v, buf0, buf1, acc, tmp, res, shared,
             sem0, sem1, rsem):
    cid = lax.axis_index("c")
    sid = lax.axis_index("s")
    wid = cid * NS + sid
    row0 = wid * ROWS_PER_W

    pltpu.sync_copy(rl_hbm, rl_v)
    rl = rl_v[...]
    cum = jnp.cumsum(rl)
    seg_start = cum - rl
    lane = lax.iota(jnp.int32, L)
    seg_end = jnp.where(lane == B - 1, N, cum)

    neg = jnp.full((L,), NEG, jnp.float32)
    for s in range(B):
        for j in range(D // L):
            acc[s, pl.ds(j * L, L)] = neg

    bufs = (buf0, buf1)
    sems = (sem0, sem1)
    copies = [None, None]
    copies[0] = pltpu.async_copy(x_hbm.at[pl.ds(row0, CHUNK)], buf0, sem0)
    for c in range(NCHUNK):
        cur = c % 2
        buf = bufs[cur]
        copies[cur].wait()
        if c + 1 < NCHUNK:
            copies[1 - cur] = pltpu.async_copy(
                x_hbm.at[pl.ds(row0 + (c + 1) * CHUNK, CHUNK)],
                bufs[1 - cur], sems[1 - cur])
        cb = row0 + c * CHUNK
        s_lo = _seg_of(cum, cb)
        s_hi = _seg_of(cum, cb + CHUNK - 1)

        def seg_body(s, _, cb=cb, buf=buf):
            lo = jnp.maximum(_lane_pick(seg_start, s), cb) - cb
            hi = jnp.minimum(_lane_pick(seg_end, s), cb + CHUNK) - cb
            n = hi - lo

            def row_body4(i, accs, buf=buf):
                r = lo + i * 4
                out = []
                for j in range(D // L):
                    t01 = jnp.maximum(buf[r, pl.ds(j * L, L)],
                                      buf[r + 1, pl.ds(j * L, L)])
                    t23 = jnp.maximum(buf[r + 2, pl.ds(j * L, L)],
                                      buf[r + 3, pl.ds(j * L, L)])
                    out.append(jnp.maximum(accs[j], jnp.maximum(t01, t23)))
                return tuple(out)

            def row_body1(r, accs, buf=buf):
                return tuple(
                    jnp.maximum(accs[j], buf[r, pl.ds(j * L, L)])
                    for j in range(D // L)
                )

            accs = lax.fori_loop(0, n // 4, row_body4, (neg,) * (D // L))
            accs = lax.fori_loop(lo + (n // 4) * 4, hi, row_body1, accs)
            rows_idx = jnp.full((L,), s, jnp.int32)
            for j in range(D // L):
                cols = lane + j * L
                old = plsc.load_gather(acc, [rows_idx, cols])
                plsc.store_scatter(
                    acc, [rows_idx, cols], jnp.maximum(old, accs[j])
                )
            return 0

        lax.fori_loop(s_lo, s_hi + 1, seg_body, 0)

    pltpu.sync_copy(acc, shared.at[sid])
    plsc.subcore_barrier()
    fanin = [pltpu.async_copy(shared.at[t, sid], tmp.at[t], rsem)
             for t in range(NS)]
    for h in fanin:
        h.wait()
    for j in range(D // L):
        m = tmp[0, pl.ds(j * L, L)]
        for t in range(1, NS):
            m = jnp.maximum(m, tmp[t, pl.ds(j * L, L)])
        res[pl.ds(j * L, L)] = m
    pltpu.sync_copy(res, out_hbm.at[cid, sid])


@jax.jit
def kernel(x, record_len):
    xs = jnp.reshape(x, (N, D))
    mesh = plsc.VectorSubcoreMesh(core_axis_name="c", subcore_axis_name="s")
    f = functools.partial(
        pl.kernel,
        out_type=jax.ShapeDtypeStruct((NC, B, D), jnp.float32),
        mesh=mesh,
        compiler_params=pltpu.CompilerParams(needs_layout_passes=False),
        scratch_types=[
            pltpu.VMEM((B,), jnp.int32),
            pltpu.VMEM((CHUNK, D), jnp.float32),
            pltpu.VMEM((CHUNK, D), jnp.float32),
            pltpu.VMEM((B, D), jnp.float32),
            pltpu.VMEM((NS, D), jnp.float32),
            pltpu.VMEM((D,), jnp.float32),
            pltpu.VMEM_SHARED((NS, B, D), jnp.float32),
            pltpu.SemaphoreType.DMA,
            pltpu.SemaphoreType.DMA,
            pltpu.SemaphoreType.DMA,
        ],
    )(_sc_body)
    part = f(xs, record_len)
    return jnp.maximum(part[0], part[1])


# speedup vs baseline: 5.3807x; 1.1246x over previous
import functools

import jax
import jax.numpy as jnp
from jax import lax
from jax.experimental import pallas as pl
from jax.experimental.pallas import tpu as pltpu
from jax.experimental.pallas import tpu_sc as plsc

N = 32768
D = 256
B = 16
L = 16
NC = 2
NS = 16
DC = D // NC
NV = DC // L
ROWS_PER_T = N // NS
CHUNK = 256
NCHUNK = ROWS_PER_T // CHUNK
NEG = float("-inf")


def _seg_of(cum, g):
    cnt = jnp.sum(jnp.where(cum <= g, 1, 0).astype(jnp.int32))
    return jnp.minimum(cnt, B - 1)


def _lane_pick(vec, s):
    lane = lax.iota(jnp.int32, L)
    return jnp.max(jnp.where(lane == s, vec, -1))


def _sc_body(x_hbm, rl_hbm, out_hbm, rl_v, buf0, buf1, acc, tmp, res, shared,
             sem0, sem1, rsem):
    cid = lax.axis_index("c")
    sid = lax.axis_index("s")
    row0 = sid * ROWS_PER_T
    col0 = cid * DC

    pltpu.sync_copy(rl_hbm, rl_v)
    rl = rl_v[...]
    cum = jnp.cumsum(rl)
    seg_start = cum - rl
    lane = lax.iota(jnp.int32, L)
    seg_end = jnp.where(lane == B - 1, N, cum)

    neg = jnp.full((L,), NEG, jnp.float32)
    for s in range(B):
        for j in range(NV):
            acc[s, pl.ds(j * L, L)] = neg

    bufs = (buf0, buf1)
    sems = (sem0, sem1)
    copies = [None, None]
    copies[0] = pltpu.async_copy(
        x_hbm.at[0, pl.ds(row0, CHUNK), pl.ds(col0, DC)], buf0, sem0)
    for c in range(NCHUNK):
        cur = c % 2
        buf = bufs[cur]
        copies[cur].wait()
        if c + 1 < NCHUNK:
            copies[1 - cur] = pltpu.async_copy(
                x_hbm.at[0, pl.ds(row0 + (c + 1) * CHUNK, CHUNK),
                         pl.ds(col0, DC)],
                bufs[1 - cur], sems[1 - cur])
        cb = row0 + c * CHUNK
        s_lo = _seg_of(cum, cb)
        s_hi = _seg_of(cum, cb + CHUNK - 1)

        def seg_body(s, _, cb=cb, buf=buf):
            lo = jnp.maximum(_lane_pick(seg_start, s), cb) - cb
            hi = jnp.minimum(_lane_pick(seg_end, s), cb + CHUNK) - cb
            n = hi - lo

            def row_body4(i, accs, buf=buf):
                r = lo + i * 4
                out = []
                for j in range(NV):
                    t01 = jnp.maximum(buf[r, pl.ds(j * L, L)],
                                      buf[r + 1, pl.ds(j * L, L)])
                    t23 = jnp.maximum(buf[r + 2, pl.ds(j * L, L)],
                                      buf[r + 3, pl.ds(j * L, L)])
                    out.append(jnp.maximum(accs[j], jnp.maximum(t01, t23)))
                return tuple(out)

            def row_body1(r, accs, buf=buf):
                return tuple(
                    jnp.maximum(accs[j], buf[r, pl.ds(j * L, L)])
                    for j in range(NV)
                )

            accs = lax.fori_loop(0, n // 4, row_body4, (neg,) * NV)
            accs = lax.fori_loop(lo + (n // 4) * 4, hi, row_body1, accs)
            rows_idx = jnp.full((L,), s, jnp.int32)
            for j in range(NV):
                cols = lane + j * L
                old = plsc.load_gather(acc, [rows_idx, cols])
                plsc.store_scatter(
                    acc, [rows_idx, cols], jnp.maximum(old, accs[j])
                )
            return 0

        lax.fori_loop(s_lo, s_hi + 1, seg_body, 0)

    pltpu.sync_copy(acc, shared.at[sid])
    plsc.subcore_barrier()
    fanin = [pltpu.async_copy(shared.at[t, sid], tmp.at[t], rsem)
             for t in range(NS)]
    for h in fanin:
        h.wait()
    for j in range(NV):
        m = tmp[0, pl.ds(j * L, L)]
        for t in range(1, NS):
            m = jnp.maximum(m, tmp[t, pl.ds(j * L, L)])
        res[pl.ds(j * L, L)] = m
    pltpu.sync_copy(res, out_hbm.at[sid, pl.ds(col0, DC)])


@jax.jit
def kernel(x, record_len):
    mesh = plsc.VectorSubcoreMesh(core_axis_name="c", subcore_axis_name="s")
    f = functools.partial(
        pl.kernel,
        out_type=jax.ShapeDtypeStruct((B, D), jnp.float32),
        mesh=mesh,
        compiler_params=pltpu.CompilerParams(needs_layout_passes=False),
        scratch_types=[
            pltpu.VMEM((B,), jnp.int32),
            pltpu.VMEM((CHUNK, DC), jnp.float32),
            pltpu.VMEM((CHUNK, DC), jnp.float32),
            pltpu.VMEM((B, DC), jnp.float32),
            pltpu.VMEM((NS, DC), jnp.float32),
            pltpu.VMEM((DC,), jnp.float32),
            pltpu.VMEM_SHARED((NS, B, DC), jnp.float32),
            pltpu.SemaphoreType.DMA,
            pltpu.SemaphoreType.DMA,
            pltpu.SemaphoreType.DMA,
        ],
    )(_sc_body)
    return f(x, record_len)


# speedup vs baseline: 6.3059x; 1.1720x over previous
import functools

import jax
import jax.numpy as jnp
from jax import lax
from jax.experimental import pallas as pl
from jax.experimental.pallas import tpu as pltpu
from jax.experimental.pallas import tpu_sc as plsc

N = 32768
D = 256
B = 16
NEG = float("-inf")

A_TC = 16384

L = 16
NC = 2
NS = 16
DC = D // NC
NV = DC // L
SC_ROWS = N - A_TC
ROWS_PER_T = SC_ROWS // NS
CHUNK = 256
NCHUNK = ROWS_PER_T // CHUNK

R_TC = 2048
NB_TC = A_TC // R_TC


def _seg_of(cum, g):
    cnt = jnp.sum(jnp.where(cum <= g, 1, 0).astype(jnp.int32))
    return jnp.minimum(cnt, B - 1)


def _lane_pick(vec, s):
    lane = lax.iota(jnp.int32, L)
    return jnp.max(jnp.where(lane == s, vec, -1))


def _sc_body(x_hbm, rl_hbm, out_hbm, rl_v, buf0, buf1, acc, tmp, res, shared,
             sem0, sem1, rsem):
    cid = lax.axis_index("c")
    sid = lax.axis_index("s")
    row0 = A_TC + sid * ROWS_PER_T
    col0 = cid * DC

    pltpu.sync_copy(rl_hbm, rl_v)
    rl = rl_v[...]
    cum = jnp.cumsum(rl)
    seg_start = cum - rl
    lane = lax.iota(jnp.int32, L)
    seg_end = jnp.where(lane == B - 1, N, cum)

    neg = jnp.full((L,), NEG, jnp.float32)
    for s in range(B):
        for j in range(NV):
            acc[s, pl.ds(j * L, L)] = neg

    bufs = (buf0, buf1)
    sems = (sem0, sem1)
    copies = [None, None]
    copies[0] = pltpu.async_copy(
        x_hbm.at[0, pl.ds(row0, CHUNK), pl.ds(col0, DC)], buf0, sem0)
    for c in range(NCHUNK):
        cur = c % 2
        buf = bufs[cur]
        copies[cur].wait()
        if c + 1 < NCHUNK:
            copies[1 - cur] = pltpu.async_copy(
                x_hbm.at[0, pl.ds(row0 + (c + 1) * CHUNK, CHUNK),
                         pl.ds(col0, DC)],
                bufs[1 - cur], sems[1 - cur])
        cb = row0 + c * CHUNK
        s_lo = _seg_of(cum, cb)
        s_hi = _seg_of(cum, cb + CHUNK - 1)

        def seg_body(s, _, cb=cb, buf=buf):
            lo = jnp.maximum(_lane_pick(seg_start, s), cb) - cb
            hi = jnp.minimum(_lane_pick(seg_end, s), cb + CHUNK) - cb
            n = hi - lo

            def row_body4(i, accs, buf=buf):
                r = lo + i * 4
                out = []
                for j in range(NV):
                    t01 = jnp.maximum(buf[r, pl.ds(j * L, L)],
                                      buf[r + 1, pl.ds(j * L, L)])
                    t23 = jnp.maximum(buf[r + 2, pl.ds(j * L, L)],
                                      buf[r + 3, pl.ds(j * L, L)])
                    out.append(jnp.maximum(accs[j], jnp.maximum(t01, t23)))
                return tuple(out)

            def row_body1(r, accs, buf=buf):
                return tuple(
                    jnp.maximum(accs[j], buf[r, pl.ds(j * L, L)])
                    for j in range(NV)
                )

            accs = lax.fori_loop(0, n // 4, row_body4, (neg,) * NV)
            accs = lax.fori_loop(lo + (n // 4) * 4, hi, row_body1, accs)
            rows_idx = jnp.full((L,), s, jnp.int32)
            for j in range(NV):
                cols = lane + j * L
                old = plsc.load_gather(acc, [rows_idx, cols])
                plsc.store_scatter(
                    acc, [rows_idx, cols], jnp.maximum(old, accs[j])
                )
            return 0

        lax.fori_loop(s_lo, s_hi + 1, seg_body, 0)

    pltpu.sync_copy(acc, shared.at[sid])
    plsc.subcore_barrier()
    fanin = [pltpu.async_copy(shared.at[t, sid], tmp.at[t], rsem)
             for t in range(NS)]
    for h in fanin:
        h.wait()
    for j in range(NV):
        m = tmp[0, pl.ds(j * L, L)]
        for t in range(1, NS):
            m = jnp.maximum(m, tmp[t, pl.ds(j * L, L)])
        res[pl.ds(j * L, L)] = m
    pltpu.sync_copy(res, out_hbm.at[sid, pl.ds(col0, DC)])


def _sc_partial(x, record_len):
    mesh = plsc.VectorSubcoreMesh(core_axis_name="c", subcore_axis_name="s")
    f = functools.partial(
        pl.kernel,
        out_type=jax.ShapeDtypeStruct((B, D), jnp.float32),
        mesh=mesh,
        compiler_params=pltpu.CompilerParams(needs_layout_passes=False),
        scratch_types=[
            pltpu.VMEM((B,), jnp.int32),
            pltpu.VMEM((CHUNK, DC), jnp.float32),
            pltpu.VMEM((CHUNK, DC), jnp.float32),
            pltpu.VMEM((B, DC), jnp.float32),
            pltpu.VMEM((NS, DC), jnp.float32),
            pltpu.VMEM((DC,), jnp.float32),
            pltpu.VMEM_SHARED((NS, B, DC), jnp.float32),
            pltpu.SemaphoreType.DMA,
            pltpu.SemaphoreType.DMA,
            pltpu.SemaphoreType.DMA,
        ],
    )(_sc_body)
    return f(x, record_len)


def _tc_body(cum_ref, x_ref, o_ref):
    i = pl.program_id(0)
    blo = i * R_TC

    @pl.when(i == 0)
    def _init():
        o_ref[...] = jnp.full((B, D), NEG, jnp.float32)

    s_lo = jnp.int32(0)
    s_hi = jnp.int32(0)
    for s in range(B):
        s_lo += jnp.where(cum_ref[s] <= blo, 1, 0).astype(jnp.int32)
        s_hi += jnp.where(cum_ref[s] <= blo + R_TC - 1, 1, 0).astype(jnp.int32)
    s_lo = jnp.minimum(s_lo, B - 1)
    s_hi = jnp.minimum(s_hi, B - 1)

    xb = x_ref[...]
    seg_iota = lax.broadcasted_iota(jnp.int32, (B, D), 0)

    def _merge(s, bm):
        o_ref[...] = jnp.maximum(
            o_ref[...], jnp.where(seg_iota == s, bm[None, :], NEG))

    def _fast():
        _merge(s_lo, jnp.max(xb, axis=0))

    def _slow():
        row_iota = lax.broadcasted_iota(jnp.int32, (R_TC, 1), 0) + blo

        def seg_body(s, _):
            start_s = jnp.where(s == 0, 0, cum_ref[jnp.maximum(s - 1, 0)])
            end_s = jnp.where(s == B - 1, N, cum_ref[s])
            mask = (row_iota >= start_s) & (row_iota < end_s)
            _merge(s, jnp.max(jnp.where(mask, xb, NEG), axis=0))
            return 0

        lax.fori_loop(s_lo, s_hi + 1, seg_body, 0)

    lax.cond(s_lo == s_hi, _fast, _slow)


def _tc_partial(xs, cum):
    grid_spec = pltpu.PrefetchScalarGridSpec(
        num_scalar_prefetch=1,
        grid=(NB_TC,),
        in_specs=[pl.BlockSpec((R_TC, D), lambda i, s: (i, 0))],
        out_specs=pl.BlockSpec((B, D), lambda i, s: (0, 0)),
    )
    return pl.pallas_call(
        _tc_body,
        grid_spec=grid_spec,
        out_shape=jax.ShapeDtypeStruct((B, D), jnp.float32),
    )(cum, xs)


@jax.jit
def kernel(x, record_len):
    xs = jnp.reshape(x, (N, D))
    cum = jnp.cumsum(record_len)
    tc = _tc_partial(xs, cum)
    sc = _sc_partial(x, record_len)
    return jnp.maximum(tc, sc)
